# baseline (device time: 6973 ns/iter reference)
import jax
import jax.numpy as jnp
from jax import lax
from jax.experimental import pallas as pl
from jax.experimental.pallas import tpu as pltpu


def kernel(x):
    m, n = x.shape
    rows = m // 128
    half = rows // 2

    def body(x_ref, out_ref, send_buf, recv_buf, send_sems, recv_sems):
        my_x = lax.axis_index("x")
        my_y = lax.axis_index("y")
        nbr = (my_x, 1 - my_y)

        barrier_sem = pltpu.get_barrier_semaphore()
        pl.semaphore_signal(
            barrier_sem, inc=1, device_id=nbr,
            device_id_type=pl.DeviceIdType.MESH,
        )
        pl.semaphore_wait(barrier_sem, 1)

        send_buf[0:half, :] = jnp.sum(
            x_ref[0 : m // 2, :].reshape(half, 128, n), axis=2
        )
        rdma_a = pltpu.make_async_remote_copy(
            src_ref=send_buf.at[0:half],
            dst_ref=recv_buf.at[0:half],
            send_sem=send_sems.at[0],
            recv_sem=recv_sems.at[0],
            device_id=nbr,
            device_id_type=pl.DeviceIdType.MESH,
        )
        rdma_a.start()

        send_buf[half:rows, :] = jnp.sum(
            x_ref[m // 2 : m, :].reshape(half, 128, n), axis=2
        )
        rdma_b = pltpu.make_async_remote_copy(
            src_ref=send_buf.at[half:rows],
            dst_ref=recv_buf.at[half:rows],
            send_sem=send_sems.at[1],
            recv_sem=recv_sems.at[1],
            device_id=nbr,
            device_id_type=pl.DeviceIdType.MESH,
        )
        rdma_b.start()

        rdma_a.wait()
        rdma_b.wait()

        out_ref[:, :] = send_buf[:, :] + recv_buf[:, :]

    out = pl.pallas_call(
        body,
        out_shape=jax.ShapeDtypeStruct((rows, 128), jnp.float32),
        in_specs=[pl.BlockSpec(memory_space=pltpu.VMEM)],
        out_specs=pl.BlockSpec(memory_space=pltpu.VMEM),
        scratch_shapes=[
            pltpu.VMEM((rows, 128), jnp.float32),
            pltpu.VMEM((rows, 128), jnp.float32),
            pltpu.SemaphoreType.DMA((2,)),
            pltpu.SemaphoreType.DMA((2,)),
        ],
        compiler_params=pltpu.CompilerParams(collective_id=0),
    )(x)
    return out.reshape(m, 1)
